# baseline (speedup 1.0000x reference)
import functools

import jax
import jax.numpy as jnp
from jax import lax
from jax.experimental import pallas as pl
from jax.experimental.pallas import tpu as pltpu
from jax.experimental.pallas import tpu_sc as plsc

N_NODES = 10000
EMBED = 128
HID = 128
FW = HID // 2
DENSE = 128
TARGET = 3
BATCH = 4096
N_EDGES = 320000

NC = 2
NS = 16
NW = NC * NS
CHUNK = 128
AGG_CH = 160
HALFC = AGG_CH // 2
EPT = AGG_CH * CHUNK
NE_PAD = NS * EPT
NPAD = 10240
NROWS = 10240
ROWS_PER_TILE = NROWS // NS

_MESH = plsc.VectorSubcoreMesh(core_axis_name="c", subcore_axis_name="s")


def _bcast_lane(wv, j):
    return lax.gather(
        wv, jnp.full((16, 1), j, jnp.int32),
        dimension_numbers=lax.GatherDimensionNumbers(
            offset_dims=(), collapsed_slice_dims=(0,), start_index_map=(0,)),
        slice_sizes=(1,),
        mode=lax.GatherScatterMode.PROMISE_IN_BOUNDS)


@functools.partial(
    pl.kernel,
    out_type=jax.ShapeDtypeStruct((NC, NPAD), jnp.float32),
    mesh=_MESH,
    compiler_params=pltpu.CompilerParams(use_tc_tiling_on_sc=False),
    scratch_types=[
        pltpu.VMEM((HALFC, CHUNK), jnp.int32),
        pltpu.VMEM((HALFC, CHUNK), jnp.float32),
        pltpu.VMEM_SHARED((NPAD,), jnp.float32),
        pltpu.SemaphoreType.DMA,
    ],
)
def _deg_sc(dst3_hbm, ew3_hbm, zero1_hbm, out_hbm, dsts_v, ews_v, acc, sem):
    c = lax.axis_index("c")
    s = lax.axis_index("s")
    pltpu.sync_copy(dst3_hbm.at[s, pl.ds(c * HALFC, HALFC)], dsts_v)
    pltpu.sync_copy(ew3_hbm.at[s, pl.ds(c * HALFC, HALFC)], ews_v)
    pltpu.sync_copy(zero1_hbm.at[pl.ds(s * 640, 640)], acc.at[pl.ds(s * 640, 640)])
    plsc.subcore_barrier()

    G = 8

    def body(i4, carry):
        descs = [
            pltpu.async_copy(ews_v.at[i4 * G + j], acc.at[dsts_v.at[i4 * G + j]],
                             sem, add=True)
            for j in range(G)
        ]
        for dsc in descs:
            dsc.wait()
        return carry

    lax.fori_loop(0, HALFC // G, body, 0)
    plsc.subcore_barrier()

    @pl.when(s == 0)
    def _():
        pltpu.sync_copy(acc, out_hbm.at[c])


@functools.partial(
    pl.kernel,
    out_type=jax.ShapeDtypeStruct((NC, NROWS, FW), jnp.float32),
    mesh=_MESH,
    compiler_params=pltpu.CompilerParams(use_tc_tiling_on_sc=False),
    scratch_types=[
        pltpu.VMEM((HALFC, CHUNK), jnp.int32),
        pltpu.VMEM((HALFC, CHUNK), jnp.int32),
        pltpu.VMEM((HALFC, CHUNK), jnp.float32),
        pltpu.VMEM((CHUNK, FW), jnp.float32),
        pltpu.VMEM((CHUNK, FW), jnp.float32),
        pltpu.VMEM((CHUNK, FW), jnp.float32),
        pltpu.VMEM((CHUNK, FW), jnp.float32),
        pltpu.VMEM_SHARED((NROWS, FW), jnp.float32),
        pltpu.SemaphoreType.DMA,
        pltpu.SemaphoreType.DMA,
        pltpu.SemaphoreType.DMA,
        pltpu.SemaphoreType.DMA,
        pltpu.SemaphoreType.DMA,
        pltpu.SemaphoreType.DMA,
        pltpu.SemaphoreType.DMA,
        pltpu.SemaphoreType.DMA,
    ],
)
def _agg_sc(y_hbm, src3_hbm, dst3_hbm, ew3_hbm, zero2_hbm, out_hbm,
            srcs_v, dsts_v, ews_v, r0, r1, r2, r3, acc,
            g0, g1, g2, g3, s0, s1, s2, s3):
    c = lax.axis_index("c")
    s = lax.axis_index("s")
    rows = [r0, r1, r2, r3]
    gsems = [g0, g1, g2, g3]
    ssems = [s0, s1, s2, s3]
    myy = y_hbm.at[c]
    pltpu.sync_copy(zero2_hbm.at[pl.ds(s * ROWS_PER_TILE, ROWS_PER_TILE), :],
                    acc.at[pl.ds(s * ROWS_PER_TILE, ROWS_PER_TILE), :])
    plsc.subcore_barrier()

    def scale(r, i):
        def scale_body(g, carry2):
            wv = ews_v[i, pl.ds(g * 16, 16)]
            for j in range(16):
                w = _bcast_lane(wv, j)
                e = g * 16 + j
                for d in range(FW // 16):
                    r[e, pl.ds(d * 16, 16)] = r[e, pl.ds(d * 16, 16)] * w
            return carry2

        lax.fori_loop(0, CHUNK // 16, scale_body, 0)

    def wait_rows(sem, r):
        pltpu.make_async_copy(myy.at[srcs_v.at[0]], r, sem).wait()

    def wait_scat(sem, r):
        pltpu.make_async_copy(r, acc.at[dsts_v.at[0]], sem).wait()

    def half_body(h, carry):
        pltpu.sync_copy(src3_hbm.at[s, pl.ds(h * HALFC, HALFC)], srcs_v)
        pltpu.sync_copy(dst3_hbm.at[s, pl.ds(h * HALFC, HALFC)], dsts_v)
        pltpu.sync_copy(ew3_hbm.at[s, pl.ds(h * HALFC, HALFC)], ews_v)
        for k in range(3):
            pltpu.async_copy(myy.at[srcs_v.at[k]], rows[k], gsems[k])

        def body(i2, carry2):
            for k in range(4):
                i = 4 * i2 + k
                wait_rows(gsems[k], rows[k])
                scale(rows[k], i)
                pltpu.async_copy(rows[k], acc.at[dsts_v.at[i]], ssems[k],
                                 add=True)
                nk = (k + 3) % 4
                if k == 0:
                    @pl.when(i2 == 0)
                    def _():
                        pltpu.async_copy(myy.at[srcs_v.at[3]], rows[3],
                                         gsems[3])

                    @pl.when((i2 > 0) & (i + 3 < HALFC))
                    def _():
                        wait_scat(ssems[nk], rows[nk])
                        pltpu.async_copy(myy.at[srcs_v.at[i + 3]], rows[nk],
                                         gsems[nk])
                else:
                    @pl.when(i + 3 < HALFC)
                    def _():
                        wait_scat(ssems[nk], rows[nk])
                        pltpu.async_copy(myy.at[srcs_v.at[i + 3]], rows[nk],
                                         gsems[nk])
            return carry2

        lax.fori_loop(0, HALFC // 4, body, 0)
        for k in range(4):
            wait_scat(ssems[k], rows[k])
        return carry

    lax.fori_loop(0, 2, half_body, 0)
    plsc.subcore_barrier()
    pltpu.sync_copy(acc.at[pl.ds(s * ROWS_PER_TILE, ROWS_PER_TILE), :],
                    out_hbm.at[c, pl.ds(s * ROWS_PER_TILE, ROWS_PER_TILE), :])


@functools.partial(
    pl.kernel,
    out_type=jax.ShapeDtypeStruct((2 * BATCH, HID), jnp.float32),
    mesh=_MESH,
    scratch_types=[
        pltpu.VMEM((CHUNK,), jnp.int32),
        pltpu.VMEM((CHUNK, HID), jnp.float32),
        pltpu.SemaphoreType.DMA,
    ],
)
def _gather_sc(x_hbm, idx_hbm, out_hbm, idx_v, rows_v, sem):
    c = lax.axis_index("c")
    s = lax.axis_index("s")
    wid = c * NS + s
    per_w = (2 * BATCH) // NW
    for j in range(per_w // CHUNK):
        base = wid * per_w + j * CHUNK
        pltpu.sync_copy(idx_hbm.at[pl.ds(base, CHUNK)], idx_v)
        pltpu.async_copy(x_hbm.at[idx_v], rows_v, sem).wait()
        pltpu.sync_copy(rows_v, out_hbm.at[pl.ds(base, CHUNK), :])


_BLK = 1000


def _split(t):
    return jnp.stack([t[:, :FW], t[:, FW:]], axis=0)


def _tc_first_body(degp_ref, emb_ref, w_ref, y_ref, dinv_ref):
    deg = jnp.sum(degp_ref[...], axis=1) + 1.0
    dinv = lax.rsqrt(deg)[:, None]
    xw = jnp.dot(emb_ref[...], w_ref[...], preferred_element_type=jnp.float32)
    y_ref[...] = _split(dinv * xw)
    dinv_ref[...] = dinv


def _tc_mid_body(agg_ref, y_ref, dinv_ref, b_ref, w_ref, ynext_ref):
    aggf = jnp.concatenate([agg_ref[0], agg_ref[1]], axis=1)
    yf = jnp.concatenate([y_ref[0], y_ref[1]], axis=1)
    x = jnp.maximum(dinv_ref[...] * (aggf + yf) + b_ref[...], 0.0)
    ynext_ref[...] = _split(dinv_ref[...] * jnp.dot(
        x, w_ref[...], preferred_element_type=jnp.float32))


def _tc_last_body(agg_ref, y_ref, dinv_ref, b_ref, x_ref):
    aggf = jnp.concatenate([agg_ref[0], agg_ref[1]], axis=1)
    yf = jnp.concatenate([y_ref[0], y_ref[1]], axis=1)
    x_ref[...] = jnp.maximum(dinv_ref[...] * (aggf + yf) + b_ref[...], 0.0)


def _tc_mlp_body(gh_ref, ga_ref, w1t_ref, w1b_ref, b1_ref, w2_ref, b2_ref,
                 w3_ref, b3_ref, out_ref):
    h = jnp.dot(gh_ref[...], w1t_ref[...], preferred_element_type=jnp.float32)
    h = h + jnp.dot(ga_ref[...], w1b_ref[...], preferred_element_type=jnp.float32)
    h = jnp.maximum(h + b1_ref[...], 0.0)
    h = jnp.maximum(
        jnp.dot(h, w2_ref[...], preferred_element_type=jnp.float32) + b2_ref[...], 0.0)
    logit = jnp.maximum(
        jnp.dot(h, w3_ref[...], preferred_element_type=jnp.float32) + b3_ref[...], 0.0)
    col = lax.broadcasted_iota(jnp.int32, logit.shape, 1)
    valid = col < TARGET
    lm = jnp.where(valid, logit, -1e30)
    m = jnp.max(lm, axis=1, keepdims=True)
    ssum = jnp.sum(jnp.where(valid, jnp.exp(lm - m), 0.0), axis=1, keepdims=True)
    out_ref[...] = logit - m - jnp.log(ssum)


def kernel(edge_index, edge_weight, home, away, emb, W1, b1, W2, b2, W3, b3,
           L1W, L1b, L2W, L2b, L3W, L3b):
    f32 = jnp.float32
    src = edge_index[0].astype(jnp.int32)
    dst = edge_index[1].astype(jnp.int32)
    ew = edge_weight.astype(f32)
    pad = NE_PAD - N_EDGES
    pad_idx = jnp.arange(pad, dtype=jnp.int32) % N_NODES
    src_p = jnp.concatenate([src, pad_idx]).reshape(NS, AGG_CH, CHUNK)
    dst_p = jnp.concatenate([dst, pad_idx]).reshape(NS, AGG_CH, CHUNK)
    ew_p = jnp.concatenate([ew, jnp.zeros((pad,), f32)]).reshape(
        NS, AGG_CH, CHUNK)
    zero1 = jnp.zeros((NPAD,), f32)
    zero2 = jnp.zeros((NROWS, FW), f32)
    idx_all = jnp.concatenate([home, away]).astype(jnp.int32)

    degp = _deg_sc(dst_p, ew_p, zero1)

    grid = (N_NODES // _BLK,)
    y1, dinv = pl.pallas_call(
        _tc_first_body,
        grid=grid,
        in_specs=[
            pl.BlockSpec((_BLK, NC), lambda i: (i, 0)),
            pl.BlockSpec((_BLK, EMBED), lambda i: (i, 0)),
            pl.BlockSpec((EMBED, HID), lambda i: (0, 0)),
        ],
        out_specs=[
            pl.BlockSpec((NC, _BLK, FW), lambda i: (0, i, 0)),
            pl.BlockSpec((_BLK, 1), lambda i: (i, 0)),
        ],
        out_shape=[
            jax.ShapeDtypeStruct((NC, N_NODES, FW), f32),
            jax.ShapeDtypeStruct((N_NODES, 1), f32),
        ],
    )(degp[:, :N_NODES].T, emb, W1)

    def mid_layer(y, b, w_next):
        aggp = _agg_sc(y, src_p, dst_p, ew_p, zero2)
        return pl.pallas_call(
            _tc_mid_body,
            grid=grid,
            in_specs=[
                pl.BlockSpec((NC, _BLK, FW), lambda i: (0, i, 0)),
                pl.BlockSpec((NC, _BLK, FW), lambda i: (0, i, 0)),
                pl.BlockSpec((_BLK, 1), lambda i: (i, 0)),
                pl.BlockSpec((1, HID), lambda i: (0, 0)),
                pl.BlockSpec((HID, HID), lambda i: (0, 0)),
            ],
            out_specs=pl.BlockSpec((NC, _BLK, FW), lambda i: (0, i, 0)),
            out_shape=jax.ShapeDtypeStruct((NC, N_NODES, FW), f32),
        )(aggp, y, dinv, b.reshape(1, HID), w_next)

    y2 = mid_layer(y1, b1, W2)
    y3 = mid_layer(y2, b2, W3)
    aggp3 = _agg_sc(y3, src_p, dst_p, ew_p, zero2)
    x3 = pl.pallas_call(
        _tc_last_body,
        grid=grid,
        in_specs=[
            pl.BlockSpec((NC, _BLK, FW), lambda i: (0, i, 0)),
            pl.BlockSpec((NC, _BLK, FW), lambda i: (0, i, 0)),
            pl.BlockSpec((_BLK, 1), lambda i: (i, 0)),
            pl.BlockSpec((1, HID), lambda i: (0, 0)),
        ],
        out_specs=pl.BlockSpec((_BLK, HID), lambda i: (i, 0)),
        out_shape=jax.ShapeDtypeStruct((N_NODES, HID), f32),
    )(aggp3, y3, dinv, b3.reshape(1, HID))

    gathered = _gather_sc(x3, idx_all)
    gh = gathered[:BATCH]
    ga = gathered[BATCH:]

    w3p = jnp.zeros((DENSE, 128), f32).at[:, :TARGET].set(L3W)
    b3p = jnp.zeros((1, 128), f32).at[0, :TARGET].set(L3b)
    mblk = 512
    mgrid = (BATCH // mblk,)
    full = pl.pallas_call(
        _tc_mlp_body,
        grid=mgrid,
        in_specs=[
            pl.BlockSpec((mblk, HID), lambda i: (i, 0)),
            pl.BlockSpec((mblk, HID), lambda i: (i, 0)),
            pl.BlockSpec((HID, DENSE), lambda i: (0, 0)),
            pl.BlockSpec((HID, DENSE), lambda i: (0, 0)),
            pl.BlockSpec((1, DENSE), lambda i: (0, 0)),
            pl.BlockSpec((DENSE, DENSE), lambda i: (0, 0)),
            pl.BlockSpec((1, DENSE), lambda i: (0, 0)),
            pl.BlockSpec((DENSE, 128), lambda i: (0, 0)),
            pl.BlockSpec((1, 128), lambda i: (0, 0)),
        ],
        out_specs=pl.BlockSpec((mblk, 128), lambda i: (i, 0)),
        out_shape=jax.ShapeDtypeStruct((BATCH, 128), f32),
    )(gh, ga, L1W[:HID], L1W[HID:], L1b.reshape(1, DENSE), L2W,
      L2b.reshape(1, DENSE), w3p, b3p)
    return full[:, :TARGET]


# speedup vs baseline: 2.2996x; 2.2996x over previous
import functools

import jax
import jax.numpy as jnp
from jax import lax
from jax.experimental import pallas as pl
from jax.experimental.pallas import tpu as pltpu
from jax.experimental.pallas import tpu_sc as plsc

N_NODES = 10000
EMBED = 128
HID = 128
DENSE = 128
TARGET = 3
BATCH = 4096
N_EDGES = 320000

NC = 2
NS = 16
NW = NC * NS
CHUNK = 112
NCHUNKS = 90
QC = 15
EPW = NCHUNKS * CHUNK
NE_PAD = NW * EPW
NPAD = 10240
NROWS = 10240
ROWS_PER_TILE = NROWS // NS

_MESH = plsc.VectorSubcoreMesh(core_axis_name="c", subcore_axis_name="s")


@functools.partial(
    pl.kernel,
    out_type=jax.ShapeDtypeStruct((NC, NPAD), jnp.float32),
    mesh=_MESH,
    scratch_types=[
        pltpu.VMEM((NCHUNKS, CHUNK), jnp.int32),
        pltpu.VMEM((NCHUNKS, CHUNK), jnp.float32),
        pltpu.VMEM_SHARED((NPAD,), jnp.float32),
        pltpu.SemaphoreType.DMA,
    ],
)
def _deg_sc(dst3_hbm, ew3_hbm, zero1_hbm, out_hbm, dsts_v, ews_v, acc, sem):
    c = lax.axis_index("c")
    s = lax.axis_index("s")
    wid = c * NS + s
    pltpu.sync_copy(dst3_hbm.at[wid], dsts_v)
    pltpu.sync_copy(ew3_hbm.at[wid], ews_v)
    pltpu.sync_copy(zero1_hbm.at[pl.ds(s * 640, 640)], acc.at[pl.ds(s * 640, 640)])
    plsc.subcore_barrier()

    G = 6

    def body(i4, carry):
        descs = [
            pltpu.async_copy(ews_v.at[i4 * G + j], acc.at[dsts_v.at[i4 * G + j]],
                             sem, add=True)
            for j in range(G)
        ]
        for dsc in descs:
            dsc.wait()
        return carry

    lax.fori_loop(0, NCHUNKS // G, body, 0)
    plsc.subcore_barrier()

    @pl.when(s == 0)
    def _():
        pltpu.sync_copy(acc, out_hbm.at[c])


@functools.partial(
    pl.kernel,
    out_type=jax.ShapeDtypeStruct((NC, NROWS, HID), jnp.float32),
    mesh=_MESH,
    scratch_types=[
        pltpu.VMEM((QC, CHUNK), jnp.int32),
        pltpu.VMEM((QC, CHUNK), jnp.int32),
        pltpu.VMEM((QC, CHUNK), jnp.float32),
        pltpu.VMEM((CHUNK, HID), jnp.float32),
        pltpu.VMEM((CHUNK, HID), jnp.float32),
        pltpu.VMEM((CHUNK, HID), jnp.float32),
        pltpu.VMEM_SHARED((NROWS, HID), jnp.float32),
        pltpu.SemaphoreType.DMA,
        pltpu.SemaphoreType.DMA,
        pltpu.SemaphoreType.DMA,
        pltpu.SemaphoreType.DMA,
        pltpu.SemaphoreType.DMA,
        pltpu.SemaphoreType.DMA,
    ],
)
def _agg_sc(y_hbm, src3_hbm, dst3_hbm, ew3_hbm, zero2_hbm, out_hbm,
            srcs_v, dsts_v, ews_v, r0, r1, r2, acc,
            g0, g1, g2, s0, s1, s2):
    c = lax.axis_index("c")
    s = lax.axis_index("s")
    wid = c * NS + s
    rows = [r0, r1, r2]
    gsems = [g0, g1, g2]
    ssems = [s0, s1, s2]
    pltpu.sync_copy(zero2_hbm.at[pl.ds(s * ROWS_PER_TILE, ROWS_PER_TILE), :],
                    acc.at[pl.ds(s * ROWS_PER_TILE, ROWS_PER_TILE), :])
    plsc.subcore_barrier()

    def scale(rows, i):
        def scale_body(g, carry2):
            wv = ews_v[i, pl.ds(g * 16, 16)]
            for j in range(16):
                w = lax.gather(
                    wv, jnp.full((16, 1), j, jnp.int32),
                    dimension_numbers=lax.GatherDimensionNumbers(
                        offset_dims=(), collapsed_slice_dims=(0,),
                        start_index_map=(0,)),
                    slice_sizes=(1,),
                    mode=lax.GatherScatterMode.PROMISE_IN_BOUNDS)
                e = g * 16 + j
                for d in range(HID // 16):
                    rows[e, pl.ds(d * 16, 16)] = rows[e, pl.ds(d * 16, 16)] * w
            return carry2

        lax.fori_loop(0, CHUNK // 16, scale_body, 0)

    def wait_rows(sem, r):
        pltpu.make_async_copy(y_hbm.at[srcs_v.at[0]], r, sem).wait()

    def wait_scat(sem, r):
        pltpu.make_async_copy(r, acc.at[dsts_v.at[0]], sem).wait()

    def round_body(h, carry):
        pltpu.sync_copy(src3_hbm.at[wid, h], srcs_v)
        pltpu.sync_copy(dst3_hbm.at[wid, h], dsts_v)
        pltpu.sync_copy(ew3_hbm.at[wid, h], ews_v)
        pltpu.async_copy(y_hbm.at[srcs_v.at[0]], rows[0], gsems[0])
        pltpu.async_copy(y_hbm.at[srcs_v.at[1]], rows[1], gsems[1])

        def body(i3, carry2):
            for k in range(3):
                i = 3 * i3 + k
                wait_rows(gsems[k], rows[k])
                scale(rows[k], i)
                pltpu.async_copy(rows[k], acc.at[dsts_v.at[i]], ssems[k],
                                 add=True)
                nk = (k + 2) % 3
                if k == 0:
                    @pl.when(i3 == 0)
                    def _():
                        pltpu.async_copy(y_hbm.at[srcs_v.at[2]], rows[2],
                                         gsems[2])

                    @pl.when((i3 > 0) & (i + 2 < QC))
                    def _():
                        wait_scat(ssems[nk], rows[nk])
                        pltpu.async_copy(y_hbm.at[srcs_v.at[i + 2]], rows[nk],
                                         gsems[nk])
                else:
                    @pl.when(i + 2 < QC)
                    def _():
                        wait_scat(ssems[nk], rows[nk])
                        pltpu.async_copy(y_hbm.at[srcs_v.at[i + 2]], rows[nk],
                                         gsems[nk])
            return carry2

        lax.fori_loop(0, QC // 3, body, 0)
        for k in range(3):
            wait_scat(ssems[k], rows[k])
        return carry

    lax.fori_loop(0, NCHUNKS // QC, round_body, 0)
    plsc.subcore_barrier()
    pltpu.sync_copy(acc.at[pl.ds(s * ROWS_PER_TILE, ROWS_PER_TILE), :],
                    out_hbm.at[c, pl.ds(s * ROWS_PER_TILE, ROWS_PER_TILE), :])


@functools.partial(
    pl.kernel,
    out_type=jax.ShapeDtypeStruct((2 * BATCH, HID), jnp.float32),
    mesh=_MESH,
    scratch_types=[
        pltpu.VMEM((128,), jnp.int32),
        pltpu.VMEM((128, HID), jnp.float32),
        pltpu.SemaphoreType.DMA,
    ],
)
def _gather_sc(x_hbm, idx_hbm, out_hbm, idx_v, rows_v, sem):
    c = lax.axis_index("c")
    s = lax.axis_index("s")
    wid = c * NS + s
    per_w = (2 * BATCH) // NW
    for j in range(per_w // 128):
        base = wid * per_w + j * 128
        pltpu.sync_copy(idx_hbm.at[pl.ds(base, 128)], idx_v)
        pltpu.async_copy(x_hbm.at[idx_v], rows_v, sem).wait()
        pltpu.sync_copy(rows_v, out_hbm.at[pl.ds(base, 128), :])


_BLK = 1000


def _tc_first_body(degp_ref, emb_ref, w_ref, y_ref, dinv_ref):
    deg = jnp.sum(degp_ref[...], axis=1) + 1.0
    dinv = lax.rsqrt(deg)[:, None]
    xw = jnp.dot(emb_ref[...], w_ref[...], preferred_element_type=jnp.float32)
    y_ref[...] = dinv * xw
    dinv_ref[...] = dinv


def _tc_mid_body(agg_ref, y_ref, dinv_ref, b_ref, w_ref, ynext_ref):
    a = agg_ref[0] + agg_ref[1] + y_ref[...]
    x = jnp.maximum(dinv_ref[...] * a + b_ref[...], 0.0)
    ynext_ref[...] = dinv_ref[...] * jnp.dot(
        x, w_ref[...], preferred_element_type=jnp.float32)


def _tc_last_body(agg_ref, y_ref, dinv_ref, b_ref, x_ref):
    a = agg_ref[0] + agg_ref[1] + y_ref[...]
    x_ref[...] = jnp.maximum(dinv_ref[...] * a + b_ref[...], 0.0)


def _tc_mlp_body(gh_ref, ga_ref, w1t_ref, w1b_ref, b1_ref, w2_ref, b2_ref,
                 w3_ref, b3_ref, out_ref):
    h = jnp.dot(gh_ref[...], w1t_ref[...], preferred_element_type=jnp.float32)
    h = h + jnp.dot(ga_ref[...], w1b_ref[...], preferred_element_type=jnp.float32)
    h = jnp.maximum(h + b1_ref[...], 0.0)
    h = jnp.maximum(
        jnp.dot(h, w2_ref[...], preferred_element_type=jnp.float32) + b2_ref[...], 0.0)
    logit = jnp.maximum(
        jnp.dot(h, w3_ref[...], preferred_element_type=jnp.float32) + b3_ref[...], 0.0)
    col = lax.broadcasted_iota(jnp.int32, logit.shape, 1)
    valid = col < TARGET
    lm = jnp.where(valid, logit, -1e30)
    m = jnp.max(lm, axis=1, keepdims=True)
    ssum = jnp.sum(jnp.where(valid, jnp.exp(lm - m), 0.0), axis=1, keepdims=True)
    out_ref[...] = logit - m - jnp.log(ssum)


def _row_specs(nrows, blk, *shapes_full):
    return [pl.BlockSpec((blk,) + s, lambda i: (i,) + (0,) * len(s))
            for s in shapes_full]


def kernel(edge_index, edge_weight, home, away, emb, W1, b1, W2, b2, W3, b3,
           L1W, L1b, L2W, L2b, L3W, L3b):
    f32 = jnp.float32
    src = edge_index[0].astype(jnp.int32)
    dst = edge_index[1].astype(jnp.int32)
    ew = edge_weight.astype(f32)
    pad = NE_PAD - N_EDGES
    pad_idx = jnp.arange(pad, dtype=jnp.int32) % N_NODES
    src_p = jnp.concatenate([src, pad_idx]).reshape(NW, NCHUNKS, CHUNK)
    dst_p = jnp.concatenate([dst, pad_idx]).reshape(NW, NCHUNKS, CHUNK)
    ew_p = jnp.concatenate([ew, jnp.zeros((pad,), f32)]).reshape(
        NW, NCHUNKS, CHUNK)
    zero1 = jnp.zeros((NPAD,), f32)
    zero2 = jnp.zeros((NROWS, HID), f32)
    idx_all = jnp.concatenate([home, away]).astype(jnp.int32)

    degp = _deg_sc(dst_p, ew_p, zero1)

    grid = (N_NODES // _BLK,)
    y1, dinv = pl.pallas_call(
        _tc_first_body,
        grid=grid,
        in_specs=[
            pl.BlockSpec((_BLK, NC), lambda i: (i, 0)),
            pl.BlockSpec((_BLK, EMBED), lambda i: (i, 0)),
            pl.BlockSpec((EMBED, HID), lambda i: (0, 0)),
        ],
        out_specs=[
            pl.BlockSpec((_BLK, HID), lambda i: (i, 0)),
            pl.BlockSpec((_BLK, 1), lambda i: (i, 0)),
        ],
        out_shape=[
            jax.ShapeDtypeStruct((N_NODES, HID), f32),
            jax.ShapeDtypeStruct((N_NODES, 1), f32),
        ],
    )(degp[:, :N_NODES].T, emb, W1)

    src_p4 = src_p.reshape(NW, NCHUNKS // QC, QC, CHUNK)
    dst_p4 = dst_p.reshape(NW, NCHUNKS // QC, QC, CHUNK)
    ew_p4 = ew_p.reshape(NW, NCHUNKS // QC, QC, CHUNK)

    def mid_layer(y, b, w_next):
        aggp = _agg_sc(y, src_p4, dst_p4, ew_p4, zero2)
        return pl.pallas_call(
            _tc_mid_body,
            grid=grid,
            in_specs=[
                pl.BlockSpec((NC, _BLK, HID), lambda i: (0, i, 0)),
                pl.BlockSpec((_BLK, HID), lambda i: (i, 0)),
                pl.BlockSpec((_BLK, 1), lambda i: (i, 0)),
                pl.BlockSpec((1, HID), lambda i: (0, 0)),
                pl.BlockSpec((HID, HID), lambda i: (0, 0)),
            ],
            out_specs=pl.BlockSpec((_BLK, HID), lambda i: (i, 0)),
            out_shape=jax.ShapeDtypeStruct((N_NODES, HID), f32),
        )(aggp, y, dinv, b.reshape(1, HID), w_next)

    y2 = mid_layer(y1, b1, W2)
    y3 = mid_layer(y2, b2, W3)
    aggp3 = _agg_sc(y3, src_p4, dst_p4, ew_p4, zero2)
    x3 = pl.pallas_call(
        _tc_last_body,
        grid=grid,
        in_specs=[
            pl.BlockSpec((NC, _BLK, HID), lambda i: (0, i, 0)),
            pl.BlockSpec((_BLK, HID), lambda i: (i, 0)),
            pl.BlockSpec((_BLK, 1), lambda i: (i, 0)),
            pl.BlockSpec((1, HID), lambda i: (0, 0)),
        ],
        out_specs=pl.BlockSpec((_BLK, HID), lambda i: (i, 0)),
        out_shape=jax.ShapeDtypeStruct((N_NODES, HID), f32),
    )(aggp3, y3, dinv, b3.reshape(1, HID))

    gathered = _gather_sc(x3, idx_all)
    gh = gathered[:BATCH]
    ga = gathered[BATCH:]

    w3p = jnp.zeros((DENSE, 128), f32).at[:, :TARGET].set(L3W)
    b3p = jnp.zeros((1, 128), f32).at[0, :TARGET].set(L3b)
    mblk = 512
    mgrid = (BATCH // mblk,)
    full = pl.pallas_call(
        _tc_mlp_body,
        grid=mgrid,
        in_specs=[
            pl.BlockSpec((mblk, HID), lambda i: (i, 0)),
            pl.BlockSpec((mblk, HID), lambda i: (i, 0)),
            pl.BlockSpec((HID, DENSE), lambda i: (0, 0)),
            pl.BlockSpec((HID, DENSE), lambda i: (0, 0)),
            pl.BlockSpec((1, DENSE), lambda i: (0, 0)),
            pl.BlockSpec((DENSE, DENSE), lambda i: (0, 0)),
            pl.BlockSpec((1, DENSE), lambda i: (0, 0)),
            pl.BlockSpec((DENSE, 128), lambda i: (0, 0)),
            pl.BlockSpec((1, 128), lambda i: (0, 0)),
        ],
        out_specs=pl.BlockSpec((mblk, 128), lambda i: (i, 0)),
        out_shape=jax.ShapeDtypeStruct((BATCH, 128), f32),
    )(gh, ga, L1W[:HID], L1W[HID:], L1b.reshape(1, DENSE), L2W,
      L2b.reshape(1, DENSE), w3p, b3p)
    return full[:, :TARGET]


# speedup vs baseline: 2.3036x; 1.0017x over previous
import functools

import jax
import jax.numpy as jnp
from jax import lax
from jax.experimental import pallas as pl
from jax.experimental.pallas import tpu as pltpu
from jax.experimental.pallas import tpu_sc as plsc

N_NODES = 10000
EMBED = 128
HID = 128
DENSE = 128
TARGET = 3
BATCH = 4096
N_EDGES = 320000

NC = 2
NS = 16
NW = NC * NS
CHUNK = 112
NCHUNKS = 90
QC = 15
EPW = NCHUNKS * CHUNK
NE_PAD = NW * EPW
NPAD = 10240
NROWS = 10240
ROWS_PER_TILE = NROWS // NS

_MESH = plsc.VectorSubcoreMesh(core_axis_name="c", subcore_axis_name="s")


@functools.partial(
    pl.kernel,
    out_type=jax.ShapeDtypeStruct((NC, NPAD), jnp.float32),
    mesh=_MESH,
    scratch_types=[
        pltpu.VMEM((NCHUNKS, CHUNK), jnp.int32),
        pltpu.VMEM((NCHUNKS, CHUNK), jnp.float32),
        pltpu.VMEM_SHARED((NPAD,), jnp.float32),
        pltpu.SemaphoreType.DMA,
    ],
)
def _deg_sc(dst3_hbm, ew3_hbm, zero1_hbm, out_hbm, dsts_v, ews_v, acc, sem):
    c = lax.axis_index("c")
    s = lax.axis_index("s")
    wid = c * NS + s
    pltpu.sync_copy(dst3_hbm.at[wid], dsts_v)
    pltpu.sync_copy(ew3_hbm.at[wid], ews_v)
    pltpu.sync_copy(zero1_hbm.at[pl.ds(s * 640, 640)], acc.at[pl.ds(s * 640, 640)])
    plsc.subcore_barrier()

    G = 6

    def body(i4, carry):
        descs = [
            pltpu.async_copy(ews_v.at[i4 * G + j], acc.at[dsts_v.at[i4 * G + j]],
                             sem, add=True)
            for j in range(G)
        ]
        for dsc in descs:
            dsc.wait()
        return carry

    lax.fori_loop(0, NCHUNKS // G, body, 0)
    plsc.subcore_barrier()

    @pl.when(s == 0)
    def _():
        pltpu.sync_copy(acc, out_hbm.at[c])


@functools.partial(
    pl.kernel,
    out_type=jax.ShapeDtypeStruct((NC, NROWS, HID), jnp.float32),
    mesh=_MESH,
    scratch_types=[
        pltpu.VMEM((QC, CHUNK), jnp.int32),
        pltpu.VMEM((QC, CHUNK), jnp.int32),
        pltpu.VMEM((QC, CHUNK), jnp.float32),
        pltpu.VMEM((CHUNK, HID), jnp.float32),
        pltpu.VMEM((CHUNK, HID), jnp.float32),
        pltpu.VMEM((CHUNK, HID), jnp.float32),
        pltpu.VMEM_SHARED((NROWS, HID), jnp.float32),
        pltpu.SemaphoreType.DMA,
        pltpu.SemaphoreType.DMA,
        pltpu.SemaphoreType.DMA,
        pltpu.SemaphoreType.DMA,
        pltpu.SemaphoreType.DMA,
        pltpu.SemaphoreType.DMA,
    ],
)
def _agg_sc(y_hbm, src3_hbm, dst3_hbm, ew3_hbm, zero2_hbm, out_hbm,
            srcs_v, dsts_v, ews_v, r0, r1, r2, acc,
            g0, g1, g2, s0, s1, s2):
    c = lax.axis_index("c")
    s = lax.axis_index("s")
    wid = c * NS + s
    rows = [r0, r1, r2]
    gsems = [g0, g1, g2]
    ssems = [s0, s1, s2]
    pltpu.sync_copy(zero2_hbm.at[pl.ds(s * ROWS_PER_TILE, ROWS_PER_TILE), :],
                    acc.at[pl.ds(s * ROWS_PER_TILE, ROWS_PER_TILE), :])
    plsc.subcore_barrier()

    def scale(rows, i):
        def scale_body(g, carry2):
            wv = ews_v[i, pl.ds(g * 16, 16)]
            for j in range(16):
                w = lax.gather(
                    wv, jnp.full((16, 1), j, jnp.int32),
                    dimension_numbers=lax.GatherDimensionNumbers(
                        offset_dims=(), collapsed_slice_dims=(0,),
                        start_index_map=(0,)),
                    slice_sizes=(1,),
                    mode=lax.GatherScatterMode.PROMISE_IN_BOUNDS)
                e = g * 16 + j
                for d in range(HID // 16):
                    rows[e, pl.ds(d * 16, 16)] = rows[e, pl.ds(d * 16, 16)] * w
            return carry2

        lax.fori_loop(0, CHUNK // 16, scale_body, 0)

    def wait_rows(sem, r):
        pltpu.make_async_copy(y_hbm.at[srcs_v.at[0]], r, sem).wait()

    def wait_scat(sem, r):
        pltpu.make_async_copy(r, acc.at[dsts_v.at[0]], sem).wait()

    def round_body(h, carry):
        pltpu.sync_copy(src3_hbm.at[wid, h], srcs_v)
        pltpu.sync_copy(dst3_hbm.at[wid, h], dsts_v)
        pltpu.sync_copy(ew3_hbm.at[wid, h], ews_v)
        pltpu.async_copy(y_hbm.at[srcs_v.at[0]], rows[0], gsems[0])
        pltpu.async_copy(y_hbm.at[srcs_v.at[1]], rows[1], gsems[1])

        def body(i3, carry2):
            for k in range(3):
                i = 3 * i3 + k
                wait_rows(gsems[k], rows[k])
                scale(rows[k], i)
                pltpu.async_copy(rows[k], acc.at[dsts_v.at[i]], ssems[k],
                                 add=True)
                nk = (k + 2) % 3
                if k == 0:
                    @pl.when(i3 == 0)
                    def _():
                        pltpu.async_copy(y_hbm.at[srcs_v.at[2]], rows[2],
                                         gsems[2])

                    @pl.when((i3 > 0) & (i + 2 < QC))
                    def _():
                        wait_scat(ssems[nk], rows[nk])
                        pltpu.async_copy(y_hbm.at[srcs_v.at[i + 2]], rows[nk],
                                         gsems[nk])
                else:
                    @pl.when(i + 2 < QC)
                    def _():
                        wait_scat(ssems[nk], rows[nk])
                        pltpu.async_copy(y_hbm.at[srcs_v.at[i + 2]], rows[nk],
                                         gsems[nk])
            return carry2

        lax.fori_loop(0, QC // 3, body, 0)
        for k in range(3):
            wait_scat(ssems[k], rows[k])
        return carry

    lax.fori_loop(0, NCHUNKS // QC, round_body, 0)
    plsc.subcore_barrier()
    pltpu.sync_copy(acc.at[pl.ds(s * ROWS_PER_TILE, ROWS_PER_TILE), :],
                    out_hbm.at[c, pl.ds(s * ROWS_PER_TILE, ROWS_PER_TILE), :])


@functools.partial(
    pl.kernel,
    out_type=jax.ShapeDtypeStruct((2 * BATCH, HID), jnp.float32),
    mesh=_MESH,
    scratch_types=[
        pltpu.VMEM((128,), jnp.int32),
        pltpu.VMEM((128, HID), jnp.float32),
        pltpu.SemaphoreType.DMA,
    ],
)
def _gather_sc(x_hbm, idx_hbm, out_hbm, idx_v, rows_v, sem):
    c = lax.axis_index("c")
    s = lax.axis_index("s")
    wid = c * NS + s
    per_w = (2 * BATCH) // NW
    for j in range(per_w // 128):
        base = wid * per_w + j * 128
        pltpu.sync_copy(idx_hbm.at[pl.ds(base, 128)], idx_v)
        pltpu.async_copy(x_hbm.at[idx_v], rows_v, sem).wait()
        pltpu.sync_copy(rows_v, out_hbm.at[pl.ds(base, 128), :])


_BLK = 1000


def _tc_first_body(degp_ref, emb_ref, w_ref, y_ref, dinv_ref):
    deg = jnp.sum(degp_ref[...], axis=1) + 1.0
    dinv = lax.rsqrt(deg)[:, None]
    xw = jnp.dot(emb_ref[...], w_ref[...], preferred_element_type=jnp.float32)
    y_ref[...] = dinv * xw
    dinv_ref[...] = dinv


def _tc_mid_body(agg_ref, y_ref, dinv_ref, b_ref, w_ref, ynext_ref):
    a = agg_ref[0] + agg_ref[1] + y_ref[...]
    x = jnp.maximum(dinv_ref[...] * a + b_ref[...], 0.0)
    ynext_ref[...] = dinv_ref[...] * jnp.dot(
        x, w_ref[...], preferred_element_type=jnp.float32)


def _tc_last_body(agg_ref, y_ref, dinv_ref, b_ref, x_ref):
    a = agg_ref[0] + agg_ref[1] + y_ref[...]
    x_ref[...] = jnp.maximum(dinv_ref[...] * a + b_ref[...], 0.0)


def _tc_mlp_body(gh_ref, ga_ref, w1t_ref, w1b_ref, b1_ref, w2_ref, b2_ref,
                 w3_ref, b3_ref, out_ref):
    h = jnp.dot(gh_ref[...], w1t_ref[...], preferred_element_type=jnp.float32)
    h = h + jnp.dot(ga_ref[...], w1b_ref[...], preferred_element_type=jnp.float32)
    h = jnp.maximum(h + b1_ref[...], 0.0)
    h = jnp.maximum(
        jnp.dot(h, w2_ref[...], preferred_element_type=jnp.float32) + b2_ref[...], 0.0)
    logit = jnp.maximum(
        jnp.dot(h, w3_ref[...], preferred_element_type=jnp.float32) + b3_ref[...], 0.0)
    col = lax.broadcasted_iota(jnp.int32, logit.shape, 1)
    valid = col < TARGET
    lm = jnp.where(valid, logit, -1e30)
    m = jnp.max(lm, axis=1, keepdims=True)
    ssum = jnp.sum(jnp.where(valid, jnp.exp(lm - m), 0.0), axis=1, keepdims=True)
    out_ref[...] = lax.slice(logit - m - jnp.log(ssum), (0, 0),
                             (logit.shape[0], TARGET))


def _row_specs(nrows, blk, *shapes_full):
    return [pl.BlockSpec((blk,) + s, lambda i: (i,) + (0,) * len(s))
            for s in shapes_full]


def kernel(edge_index, edge_weight, home, away, emb, W1, b1, W2, b2, W3, b3,
           L1W, L1b, L2W, L2b, L3W, L3b):
    f32 = jnp.float32
    src = edge_index[0].astype(jnp.int32)
    dst = edge_index[1].astype(jnp.int32)
    ew = edge_weight.astype(f32)
    pad = NE_PAD - N_EDGES
    pad_idx = jnp.arange(pad, dtype=jnp.int32) % N_NODES
    src_p = jnp.concatenate([src, pad_idx]).reshape(NW, NCHUNKS, CHUNK)
    dst_p = jnp.concatenate([dst, pad_idx]).reshape(NW, NCHUNKS, CHUNK)
    ew_p = jnp.concatenate([ew, jnp.zeros((pad,), f32)]).reshape(
        NW, NCHUNKS, CHUNK)
    zero1 = jnp.zeros((NPAD,), f32)
    zero2 = jnp.zeros((NROWS, HID), f32)
    idx_all = jnp.concatenate([home, away]).astype(jnp.int32)

    degp = _deg_sc(dst_p, ew_p, zero1)

    grid = (N_NODES // _BLK,)
    y1, dinv = pl.pallas_call(
        _tc_first_body,
        grid=grid,
        in_specs=[
            pl.BlockSpec((_BLK, NC), lambda i: (i, 0)),
            pl.BlockSpec((_BLK, EMBED), lambda i: (i, 0)),
            pl.BlockSpec((EMBED, HID), lambda i: (0, 0)),
        ],
        out_specs=[
            pl.BlockSpec((_BLK, HID), lambda i: (i, 0)),
            pl.BlockSpec((_BLK, 1), lambda i: (i, 0)),
        ],
        out_shape=[
            jax.ShapeDtypeStruct((N_NODES, HID), f32),
            jax.ShapeDtypeStruct((N_NODES, 1), f32),
        ],
    )(degp[:, :N_NODES].T, emb, W1)

    src_p4 = src_p.reshape(NW, NCHUNKS // QC, QC, CHUNK)
    dst_p4 = dst_p.reshape(NW, NCHUNKS // QC, QC, CHUNK)
    ew_p4 = ew_p.reshape(NW, NCHUNKS // QC, QC, CHUNK)

    def mid_layer(y, b, w_next):
        aggp = _agg_sc(y, src_p4, dst_p4, ew_p4, zero2)
        return pl.pallas_call(
            _tc_mid_body,
            grid=grid,
            in_specs=[
                pl.BlockSpec((NC, _BLK, HID), lambda i: (0, i, 0)),
                pl.BlockSpec((_BLK, HID), lambda i: (i, 0)),
                pl.BlockSpec((_BLK, 1), lambda i: (i, 0)),
                pl.BlockSpec((1, HID), lambda i: (0, 0)),
                pl.BlockSpec((HID, HID), lambda i: (0, 0)),
            ],
            out_specs=pl.BlockSpec((_BLK, HID), lambda i: (i, 0)),
            out_shape=jax.ShapeDtypeStruct((N_NODES, HID), f32),
        )(aggp, y, dinv, b.reshape(1, HID), w_next)

    y2 = mid_layer(y1, b1, W2)
    y3 = mid_layer(y2, b2, W3)
    aggp3 = _agg_sc(y3, src_p4, dst_p4, ew_p4, zero2)
    x3 = pl.pallas_call(
        _tc_last_body,
        grid=grid,
        in_specs=[
            pl.BlockSpec((NC, _BLK, HID), lambda i: (0, i, 0)),
            pl.BlockSpec((_BLK, HID), lambda i: (i, 0)),
            pl.BlockSpec((_BLK, 1), lambda i: (i, 0)),
            pl.BlockSpec((1, HID), lambda i: (0, 0)),
        ],
        out_specs=pl.BlockSpec((_BLK, HID), lambda i: (i, 0)),
        out_shape=jax.ShapeDtypeStruct((N_NODES, HID), f32),
    )(aggp3, y3, dinv, b3.reshape(1, HID))

    gathered = _gather_sc(x3, idx_all)
    gh = gathered[:BATCH]
    ga = gathered[BATCH:]

    w3p = jnp.zeros((DENSE, 128), f32).at[:, :TARGET].set(L3W)
    b3p = jnp.zeros((1, 128), f32).at[0, :TARGET].set(L3b)
    mblk = 512
    mgrid = (BATCH // mblk,)
    full = pl.pallas_call(
        _tc_mlp_body,
        grid=mgrid,
        in_specs=[
            pl.BlockSpec((mblk, HID), lambda i: (i, 0)),
            pl.BlockSpec((mblk, HID), lambda i: (i, 0)),
            pl.BlockSpec((HID, DENSE), lambda i: (0, 0)),
            pl.BlockSpec((HID, DENSE), lambda i: (0, 0)),
            pl.BlockSpec((1, DENSE), lambda i: (0, 0)),
            pl.BlockSpec((DENSE, DENSE), lambda i: (0, 0)),
            pl.BlockSpec((1, DENSE), lambda i: (0, 0)),
            pl.BlockSpec((DENSE, 128), lambda i: (0, 0)),
            pl.BlockSpec((1, 128), lambda i: (0, 0)),
        ],
        out_specs=pl.BlockSpec((mblk, TARGET), lambda i: (i, 0)),
        out_shape=jax.ShapeDtypeStruct((BATCH, TARGET), f32),
    )(gh, ga, L1W[:HID], L1W[HID:], L1b.reshape(1, DENSE), L2W,
      L2b.reshape(1, DENSE), w3p, b3p)
    return full
